# Initial kernel scaffold; baseline (speedup 1.0000x reference)
import jax
import jax.numpy as jnp
from jax.experimental import pallas as pl


def kernel(seq):
    raise NotImplementedError("write your pallas kernel here")


# speedup vs baseline: 2.3022x; 2.3022x over previous
import functools

import jax
import jax.numpy as jnp
from jax import lax
from jax.experimental import pallas as pl
from jax.experimental.pallas import tpu as pltpu
from jax.experimental.pallas import tpu_sc as plsc

N = 4194304
NUM_CORES = 2
NUM_SUBCORES = 16
NUM_WORKERS = NUM_CORES * NUM_SUBCORES
TOKENS_PER_WORKER = N // NUM_WORKERS
CHUNK = 16384
NUM_CHUNKS = TOKENS_PER_WORKER // CHUNK
LANES = 16

_mesh = plsc.VectorSubcoreMesh(core_axis_name="c", subcore_axis_name="s")


@functools.partial(
    pl.kernel,
    mesh=_mesh,
    out_type=jax.ShapeDtypeStruct((4, N), jnp.float32),
    scratch_types=[
        pltpu.VMEM((CHUNK,), jnp.int32),
        pltpu.VMEM((4, CHUNK), jnp.float32),
    ],
)
def _seq2tensor_sc(seq_hbm, out_hbm, seq_v, out_v):
    wid = lax.axis_index("s") * NUM_CORES + lax.axis_index("c")
    base = wid * TOKENS_PER_WORKER

    def chunk_body(ci, carry):
        off = base + ci * CHUNK
        pltpu.sync_copy(seq_hbm.at[pl.ds(off, CHUNK)], seq_v)

        def vec_body(i, carry2):
            s = seq_v[pl.ds(i * LANES, LANES)]
            bg = jnp.where(s == 4, 0.25, 0.0).astype(jnp.float32)
            for c in range(4):
                out_v[c, pl.ds(i * LANES, LANES)] = jnp.where(s == c, 1.0, bg)
            return carry2

        lax.fori_loop(0, CHUNK // LANES, vec_body, 0)
        for c in range(4):
            pltpu.sync_copy(out_v.at[c], out_hbm.at[c, pl.ds(off, CHUNK)])
        return carry

    lax.fori_loop(0, NUM_CHUNKS, chunk_body, 0)


def kernel(seq):
    return _seq2tensor_sc(seq)
